# baseline (speedup 1.0000x reference)
import functools
import math

import jax
import jax.numpy as jnp
from jax import lax
from jax.experimental import pallas as pl
from jax.experimental.pallas import tpu as pltpu
from jax.experimental.pallas import tpu_sc as plsc

VOCAB = 1_000_000
DIM = 64
NTOK = 4096 * 200
NC, NS = 2, 16
NW = NC * NS
PER_W = NTOK // NW
CHUNK = 128
NCH = PER_W // CHUNK
SCALE = math.sqrt(DIM)

_mesh = plsc.VectorSubcoreMesh(
    core_axis_name="c", subcore_axis_name="s", num_cores=NC, num_subcores=NS
)


@functools.partial(
    pl.kernel,
    out_type=jax.ShapeDtypeStruct((NTOK, DIM), jnp.float32),
    mesh=_mesh,
    compiler_params=pltpu.CompilerParams(use_tc_tiling_on_sc=False),
    scratch_types=[
        pltpu.VMEM((NCH, CHUNK), jnp.int32),
        pltpu.VMEM((CHUNK, DIM), jnp.float32),
        pltpu.SemaphoreType.DMA,
    ],
)
def _embed(ids_hbm, tab_hbm, out_hbm, idx_v, rows_v, sem):
    wid = lax.axis_index("s") * NC + lax.axis_index("c")
    base = wid * PER_W
    pltpu.sync_copy(ids_hbm.at[wid], idx_v)

    def chunk_body(c, carry):
        pltpu.async_copy(tab_hbm.at[idx_v.at[c]], rows_v, sem).wait()

        def row_body(r, carry2):
            for j in range(DIM // 16):
                sl = pl.ds(16 * j, 16)
                rows_v[r, sl] = rows_v[r, sl] * SCALE
            return carry2

        lax.fori_loop(0, CHUNK, row_body, 0, unroll=4)
        pltpu.sync_copy(rows_v, out_hbm.at[pl.ds(base + c * CHUNK, CHUNK)])
        return carry

    lax.fori_loop(0, NCH, chunk_body, 0)


def kernel(token_ids_batch, embeddings_table):
    b, s = token_ids_batch.shape
    ids = token_ids_batch.astype(jnp.int32).reshape(NW, NCH, CHUNK)
    out = _embed(ids, embeddings_table)
    return out.reshape(b, s, DIM)


# speedup vs baseline: 1.0546x; 1.0546x over previous
import functools
import math

import jax
import jax.numpy as jnp
from jax import lax
from jax.experimental import pallas as pl
from jax.experimental.pallas import tpu as pltpu
from jax.experimental.pallas import tpu_sc as plsc

VOCAB = 1_000_000
DIM = 64
NTOK = 4096 * 200
NC, NS = 2, 16
NW = NC * NS
PER_W = NTOK // NW
CHUNK = 128
NCH = PER_W // CHUNK
NBUF = 4
NROUND = NCH // NBUF
SCALE = math.sqrt(DIM)

_mesh = plsc.VectorSubcoreMesh(
    core_axis_name="c", subcore_axis_name="s", num_cores=NC, num_subcores=NS
)


@functools.partial(
    pl.kernel,
    out_type=jax.ShapeDtypeStruct((NTOK, DIM), jnp.float32),
    mesh=_mesh,
    compiler_params=pltpu.CompilerParams(use_tc_tiling_on_sc=False),
    scratch_types=[
        pltpu.VMEM((NCH, CHUNK), jnp.int32),
        pltpu.VMEM((NBUF, CHUNK, DIM), jnp.float32),
        pltpu.VMEM((NBUF, CHUNK, DIM), jnp.float32),
        [pltpu.SemaphoreType.DMA] * NBUF,
        [pltpu.SemaphoreType.DMA] * NBUF,
    ],
)
def _embed(ids_hbm, tab_hbm, out_hbm, idx_v, gbuf, obuf, gsem, osem):
    wid = lax.axis_index("s") * NC + lax.axis_index("c")
    base = wid * PER_W
    pltpu.sync_copy(ids_hbm.at[wid], idx_v)

    for b in range(NBUF):
        pltpu.async_copy(tab_hbm.at[idx_v.at[b]], gbuf.at[b], gsem[b])

    def round_body(t, carry):
        for b in range(NBUF):
            c = t * NBUF + b
            pltpu.make_async_copy(
                tab_hbm.at[pl.ds(0, CHUNK)], gbuf.at[b], gsem[b]
            ).wait()

            @pl.when(t > 0)
            def _():
                pltpu.make_async_copy(
                    obuf.at[b], out_hbm.at[pl.ds(0, CHUNK)], osem[b]
                ).wait()

            def row_body(r, cr):
                for j in range(DIM // 16):
                    sl = pl.ds(16 * j, 16)
                    obuf[b, r, sl] = gbuf[b, r, sl] * SCALE
                return cr

            lax.fori_loop(0, CHUNK, row_body, 0, unroll=4)

            pltpu.async_copy(
                obuf.at[b], out_hbm.at[pl.ds(base + c * CHUNK, CHUNK)], osem[b]
            )

            @pl.when(t < NROUND - 1)
            def _():
                pltpu.async_copy(
                    tab_hbm.at[idx_v.at[c + NBUF]], gbuf.at[b], gsem[b]
                )
        return carry

    lax.fori_loop(0, NROUND, round_body, 0)

    for b in range(NBUF):
        pltpu.make_async_copy(
            obuf.at[b], out_hbm.at[pl.ds(0, CHUNK)], osem[b]
        ).wait()


def kernel(token_ids_batch, embeddings_table):
    b, s = token_ids_batch.shape
    ids = token_ids_batch.astype(jnp.int32).reshape(NW, NCH, CHUNK)
    out = _embed(ids, embeddings_table)
    return out.reshape(b, s, DIM)
